# Initial kernel scaffold; baseline (speedup 1.0000x reference)
import jax
import jax.numpy as jnp
from jax.experimental import pallas as pl


def kernel(x, edge_index, W1, b1, W2, b2, W3, b3):
    raise NotImplementedError("write your pallas kernel here")


# speedup vs baseline: 13.6081x; 13.6081x over previous
import functools

import jax
import jax.numpy as jnp
from jax import lax
from jax.experimental import pallas as pl
from jax.experimental.pallas import tpu as pltpu
from jax.experimental.pallas import tpu_sc as plsc

N_NODES = 10000
N_EDGES = 320000
N_FEAT = 128
N_CLASS = 64

N_PAD = 10240
NW = 32
CW = 128
NCHUNK = 79
EPT = NCHUNK * CW
E_PAD = NW * EPT
RPT = N_PAD // 16

_mesh = plsc.VectorSubcoreMesh(core_axis_name="c", subcore_axis_name="s")


def _agg_body(g_hbm, src_hbm, dst_hbm, out_hbm, src_v, dst_v, rows_v, acc_sh):
    cid = lax.axis_index("c")
    sid = lax.axis_index("s")
    tid = cid * 16 + sid
    pltpu.sync_copy(src_hbm.at[tid], src_v)
    pltpu.sync_copy(dst_hbm.at[tid], dst_v)

    def zrow(r, carry):
        for k in range(N_FEAT // 16):
            rows_v[r, pl.ds(k * 16, 16)] = jnp.zeros((16,), jnp.float32)
        return carry

    lax.fori_loop(0, CW, zrow, 0)
    for k in range(RPT // CW):
        pltpu.sync_copy(rows_v, acc_sh.at[pl.ds(sid * RPT + k * CW, CW)])
    plsc.subcore_barrier()

    def step(j, carry):
        pltpu.sync_copy(g_hbm.at[src_v.at[j]], rows_v)
        pltpu.sync_copy(rows_v, acc_sh.at[dst_v.at[j]], add=True)
        return carry

    lax.fori_loop(0, NCHUNK, step, 0)
    plsc.subcore_barrier()
    pltpu.sync_copy(acc_sh.at[pl.ds(sid * RPT, RPT)],
                    out_hbm.at[cid, pl.ds(sid * RPT, RPT)])


_edge_agg = pl.kernel(
    _agg_body,
    out_type=jax.ShapeDtypeStruct((2, N_PAD, N_FEAT), jnp.float32),
    mesh=_mesh,
    scratch_types=[
        pltpu.VMEM((NCHUNK, CW), jnp.int32),
        pltpu.VMEM((NCHUNK, CW), jnp.int32),
        pltpu.VMEM((CW, N_FEAT), jnp.float32),
        pltpu.VMEM_SHARED((N_PAD, N_FEAT), jnp.float32),
    ],
)


def _deg_body(dst_hbm, out_hbm, dst_v, ones_v, zero_v, deg_sh):
    cid = lax.axis_index("c")
    sid = lax.axis_index("s")
    tid = cid * 16 + sid
    pltpu.sync_copy(dst_hbm.at[tid], dst_v)

    def fill(r, carry):
        ones_v[pl.ds(r * 16, 16)] = jnp.ones((16,), jnp.float32)
        zero_v[pl.ds(r * 16, 16)] = jnp.zeros((16,), jnp.float32)
        return carry

    lax.fori_loop(0, CW // 16, fill, 0)

    def zfill(r, carry):
        zero_v[pl.ds(r * 16, 16)] = jnp.zeros((16,), jnp.float32)
        return carry

    lax.fori_loop(CW // 16, RPT // 16, zfill, 0)
    pltpu.sync_copy(zero_v, deg_sh.at[pl.ds(sid * RPT, RPT)])
    plsc.subcore_barrier()

    def step(j, carry):
        pltpu.sync_copy(ones_v, deg_sh.at[dst_v.at[j]], add=True)
        return carry

    lax.fori_loop(0, NCHUNK, step, 0)
    plsc.subcore_barrier()
    pltpu.sync_copy(deg_sh.at[pl.ds(sid * RPT, RPT)],
                    out_hbm.at[cid, pl.ds(sid * RPT, RPT)])


_deg_count = pl.kernel(
    _deg_body,
    out_type=jax.ShapeDtypeStruct((2, N_PAD), jnp.float32),
    mesh=_mesh,
    scratch_types=[
        pltpu.VMEM((NCHUNK, CW), jnp.int32),
        pltpu.VMEM((CW,), jnp.float32),
        pltpu.VMEM((RPT,), jnp.float32),
        pltpu.VMEM_SHARED((N_PAD,), jnp.float32),
    ],
)

_BR = 2000


def _b1_body(x_ref, w_ref, degp_ref, g_ref, dinv_ref):
    deg = degp_ref[0] + degp_ref[1] + 1.0
    dinv = lax.rsqrt(deg)
    dinv_ref[...] = dinv
    g_ref[...] = jnp.dot(x_ref[...], w_ref[...],
                         preferred_element_type=jnp.float32) * dinv


_b1 = pl.pallas_call(
    _b1_body,
    grid=(N_NODES // _BR,),
    in_specs=[
        pl.BlockSpec((_BR, N_FEAT), lambda i: (i, 0)),
        pl.BlockSpec((N_FEAT, N_FEAT), lambda i: (0, 0)),
        pl.BlockSpec((2, _BR, 1), lambda i: (0, i, 0)),
    ],
    out_specs=[
        pl.BlockSpec((_BR, N_FEAT), lambda i: (i, 0)),
        pl.BlockSpec((_BR, 1), lambda i: (i, 0)),
    ],
    out_shape=[
        jax.ShapeDtypeStruct((N_NODES, N_FEAT), jnp.float32),
        jax.ShapeDtypeStruct((N_NODES, 1), jnp.float32),
    ],
)


def _b2_body(acc_ref, g_ref, dinv_ref, b_ref, w_ref, out_ref):
    s = acc_ref[0] + acc_ref[1] + g_ref[...]
    h = jnp.maximum(dinv_ref[...] * s + b_ref[...], 0.0)
    out_ref[...] = jnp.dot(h, w_ref[...],
                           preferred_element_type=jnp.float32) * dinv_ref[...]


_b2 = pl.pallas_call(
    _b2_body,
    grid=(N_NODES // _BR,),
    in_specs=[
        pl.BlockSpec((2, _BR, N_FEAT), lambda i: (0, i, 0)),
        pl.BlockSpec((_BR, N_FEAT), lambda i: (i, 0)),
        pl.BlockSpec((_BR, 1), lambda i: (i, 0)),
        pl.BlockSpec((1, N_FEAT), lambda i: (0, 0)),
        pl.BlockSpec((N_FEAT, N_FEAT), lambda i: (0, 0)),
    ],
    out_specs=pl.BlockSpec((_BR, N_FEAT), lambda i: (i, 0)),
    out_shape=jax.ShapeDtypeStruct((N_NODES, N_FEAT), jnp.float32),
)


def _b3_body(acc_ref, g_ref, dinv_ref, b_ref, w_ref, b3_ref, out_ref):
    s = acc_ref[0] + acc_ref[1] + g_ref[...]
    h = jnp.maximum(dinv_ref[...] * s + b_ref[...], 0.0)
    out_ref[...] = jnp.dot(h, w_ref[...],
                           preferred_element_type=jnp.float32) + b3_ref[...]


_b3 = pl.pallas_call(
    _b3_body,
    grid=(N_NODES // _BR,),
    in_specs=[
        pl.BlockSpec((2, _BR, N_FEAT), lambda i: (0, i, 0)),
        pl.BlockSpec((_BR, N_FEAT), lambda i: (i, 0)),
        pl.BlockSpec((_BR, 1), lambda i: (i, 0)),
        pl.BlockSpec((1, N_FEAT), lambda i: (0, 0)),
        pl.BlockSpec((N_FEAT, N_CLASS), lambda i: (0, 0)),
        pl.BlockSpec((1, N_CLASS), lambda i: (0, 0)),
    ],
    out_specs=pl.BlockSpec((_BR, N_CLASS), lambda i: (i, 0)),
    out_shape=jax.ShapeDtypeStruct((N_NODES, N_CLASS), jnp.float32),
)


def kernel(x, edge_index, W1, b1, W2, b2, W3, b3):
    src = edge_index[0].astype(jnp.int32)
    dst = edge_index[1].astype(jnp.int32)
    n_extra = E_PAD - N_EDGES
    src_p = jnp.concatenate(
        [src, jnp.zeros((n_extra,), jnp.int32)]).reshape(NW, NCHUNK, CW)
    dst_p = jnp.concatenate(
        [dst, jnp.full((n_extra,), N_NODES, jnp.int32)]).reshape(NW, NCHUNK, CW)

    degp = _deg_count(dst_p)
    g1, dinv = _b1(x, W1, degp[..., None])
    acc1 = _edge_agg(g1, src_p, dst_p)
    g2 = _b2(acc1, g1, dinv, b1.reshape(1, -1), W2)
    acc2 = _edge_agg(g2, src_p, dst_p)
    out = _b3(acc2, g2, dinv, b2.reshape(1, -1), W3, b3.reshape(1, -1))
    return out
